# Initial kernel scaffold; baseline (speedup 1.0000x reference)
import functools

import jax
import jax.numpy as jnp
from jax import lax
from jax.experimental import pallas as pl
from jax.experimental.pallas import tpu as pltpu
from jax.experimental.pallas import tpu_sc as plsc

L1_REG = 0.001
L2_REG = 0.001

N_IDX = 5 * 4096 * 26
D = 32
NW = 32
PER_W = N_IDX // NW
CHUNK = 128
N_CHUNKS = PER_W // CHUNK
G = 13
N_GROUPS = N_CHUNKS // G
GROUP_ROWS = G * CHUNK


def _gather_sc(W, Xf):
    mesh = plsc.VectorSubcoreMesh(core_axis_name="c", subcore_axis_name="s")

    @functools.partial(
        pl.kernel,
        mesh=mesh,
        out_type=jax.ShapeDtypeStruct((N_IDX, D), jnp.float32),
        scratch_types=[
            pltpu.VMEM((N_CHUNKS, CHUNK), jnp.int32),
            pltpu.VMEM((GROUP_ROWS, D), jnp.float32),
            pltpu.SemaphoreType.DMA,
        ],
    )
    def k(w_hbm, x_hbm, out_hbm, idx_v, rows_v, sem):
        nc = 2
        wid = lax.axis_index("s") * nc + lax.axis_index("c")
        base = wid * PER_W
        pltpu.sync_copy(x_hbm.at[wid], idx_v)

        def body(g, carry):
            cps = []
            for j in range(G):
                cps.append(pltpu.async_copy(
                    w_hbm.at[idx_v.at[g * G + j]],
                    rows_v.at[pl.ds(j * CHUNK, CHUNK)],
                    sem,
                ))
            for cp in cps:
                cp.wait()
            pltpu.sync_copy(
                rows_v, out_hbm.at[pl.ds(base + g * GROUP_ROWS, GROUP_ROWS)])
            return carry

        lax.fori_loop(0, N_GROUPS, body, 0)

    return k(W, Xf)


def _penalty_block(w_ref, out_ref):
    i = pl.program_id(0)
    x = w_ref[...]
    part = (L2_REG * 0.5) * jnp.sum(x * x) + L1_REG * jnp.sum(jnp.abs(x))

    @pl.when(i == 0)
    def _():
        out_ref[0, 0] = 0.0

    out_ref[0, 0] += part


def _penalty_tc(W):
    Wr = W.reshape(125000, 256)
    out = pl.pallas_call(
        _penalty_block,
        grid=(125,),
        in_specs=[pl.BlockSpec((1000, 256), lambda i: (i, 0))],
        out_specs=pl.BlockSpec(memory_space=pltpu.SMEM),
        out_shape=jax.ShapeDtypeStruct((1, 1), jnp.float32),
    )(Wr)
    return out[0, 0]


def kernel(X, W):
    n_samples, n_batch, input_dim = X.shape
    Xf = X.reshape(NW, N_CHUNKS, CHUNK)
    rows = _gather_sc(W, Xf)
    Net = rows.reshape(n_samples, n_batch, input_dim * D)
    penalty = _penalty_tc(W)
    return Net, penalty


# speedup vs baseline: 1.4202x; 1.4202x over previous
import functools

import jax
import jax.numpy as jnp
from jax import lax
from jax.experimental import pallas as pl
from jax.experimental.pallas import tpu as pltpu
from jax.experimental.pallas import tpu_sc as plsc

L1_REG = 0.001
L2_REG = 0.001

N_IDX = 5 * 4096 * 26
D = 32
NW = 32
PER_W = N_IDX // NW
CHUNK = 128
N_CHUNKS = PER_W // CHUNK
G = 13
N_GROUPS = N_CHUNKS // G
GROUP_ROWS = G * CHUNK


def _gather_sc(W, Xf):
    mesh = plsc.VectorSubcoreMesh(core_axis_name="c", subcore_axis_name="s")

    @functools.partial(
        pl.kernel,
        mesh=mesh,
        out_type=jax.ShapeDtypeStruct((N_IDX, D), jnp.float32),
        scratch_types=[
            pltpu.VMEM((N_CHUNKS, CHUNK), jnp.int32),
            pltpu.VMEM((GROUP_ROWS, D), jnp.float32),
            pltpu.SemaphoreType.DMA,
        ],
        compiler_params=pltpu.CompilerParams(use_tc_tiling_on_sc=False),
    )
    def k(w_hbm, x_hbm, out_hbm, idx_v, rows_v, sem):
        nc = 2
        wid = lax.axis_index("s") * nc + lax.axis_index("c")
        base = wid * PER_W
        pltpu.sync_copy(x_hbm.at[wid], idx_v)

        def body(g, carry):
            cps = []
            for j in range(G):
                cps.append(pltpu.async_copy(
                    w_hbm.at[idx_v.at[g * G + j]],
                    rows_v.at[pl.ds(j * CHUNK, CHUNK)],
                    sem,
                ))
            for cp in cps:
                cp.wait()
            pltpu.sync_copy(
                rows_v, out_hbm.at[pl.ds(base + g * GROUP_ROWS, GROUP_ROWS)])
            return carry

        lax.fori_loop(0, N_GROUPS, body, 0)

    return k(W, Xf)


def _penalty_block(w_ref, out_ref):
    i = pl.program_id(0)
    x = w_ref[...]
    part = (L2_REG * 0.5) * jnp.sum(x * x) + L1_REG * jnp.sum(jnp.abs(x))

    @pl.when(i == 0)
    def _():
        out_ref[0, 0] = 0.0

    out_ref[0, 0] += part


def _penalty_tc(W):
    Wr = W.reshape(125000, 256)
    out = pl.pallas_call(
        _penalty_block,
        grid=(125,),
        in_specs=[pl.BlockSpec((1000, 256), lambda i: (i, 0))],
        out_specs=pl.BlockSpec(memory_space=pltpu.SMEM),
        out_shape=jax.ShapeDtypeStruct((1, 1), jnp.float32),
    )(Wr)
    return out[0, 0]


def kernel(X, W):
    n_samples, n_batch, input_dim = X.shape
    Xf = X.reshape(NW, N_CHUNKS, CHUNK)
    rows = _gather_sc(W, Xf)
    Net = rows.reshape(n_samples, n_batch, input_dim * D)
    penalty = _penalty_tc(W)
    return Net, penalty
